# baseline (speedup 1.0000x reference)
import jax
import jax.numpy as jnp
from jax.experimental import pallas as pl

_GRID = 10


def _copy_body(x_ref, e_ref, ox_ref, oe_ref):
    ox_ref[...] = x_ref[...]
    oe_ref[...] = e_ref[...]


def kernel(x, edge_index, edge_attr):
    del edge_index
    n_nodes, d_feat = x.shape
    n_edges, d_edge = edge_attr.shape
    e2 = edge_attr.reshape(n_edges * d_edge // 128, 128)
    bx = n_nodes // _GRID
    be = e2.shape[0] // _GRID
    out = pl.pallas_call(
        _copy_body,
        grid=(_GRID,),
        in_specs=[
            pl.BlockSpec((bx, d_feat), lambda i: (i, 0)),
            pl.BlockSpec((be, 128), lambda i: (i, 0)),
        ],
        out_specs=[
            pl.BlockSpec((bx, d_feat), lambda i: (i, 0)),
            pl.BlockSpec((be, 128), lambda i: (i, 0)),
        ],
        out_shape=[
            jax.ShapeDtypeStruct(x.shape, x.dtype),
            jax.ShapeDtypeStruct(e2.shape, e2.dtype),
        ],
    )(x, e2)
    return (out[0], out[1].reshape(n_edges, d_edge))


# speedup vs baseline: 1.0658x; 1.0658x over previous
import jax
import jax.numpy as jnp
from jax.experimental import pallas as pl
from jax.experimental.pallas import tpu as pltpu

_NCHUNK = 10


def _copy_body(x_hbm, e_hbm, ox_hbm, oe_hbm, xbuf, ebuf, sx, se_in, se_out):
    n_edges = e_hbm.shape[0]
    ce = n_edges // _NCHUNK

    def e_in(k, slot):
        return pltpu.make_async_copy(
            e_hbm.at[pl.ds(k * ce, ce), :], ebuf.at[slot], se_in.at[slot])

    def e_out(k, slot):
        return pltpu.make_async_copy(
            ebuf.at[slot], oe_hbm.at[pl.ds(k * ce, ce), :], se_out.at[slot])

    x_in = pltpu.make_async_copy(x_hbm, xbuf, sx)
    x_in.start()
    e_in(0, 0).start()
    for k in range(_NCHUNK):
        slot = k % 2
        e_in(k, slot).wait()
        e_out(k, slot).start()
        if k + 1 < _NCHUNK:
            if k >= 1:
                e_out(k - 1, 1 - slot).wait()
            e_in(k + 1, 1 - slot).start()
    x_in.wait()
    x_out = pltpu.make_async_copy(xbuf, ox_hbm, sx)
    x_out.start()
    if _NCHUNK >= 2:
        e_out(_NCHUNK - 2, _NCHUNK % 2).wait()
    e_out(_NCHUNK - 1, (_NCHUNK - 1) % 2).wait()
    x_out.wait()


def kernel(x, edge_index, edge_attr):
    del edge_index
    n_edges, d_edge = edge_attr.shape
    hbm = pl.BlockSpec(memory_space=pltpu.MemorySpace.HBM)
    out = pl.pallas_call(
        _copy_body,
        in_specs=[hbm, hbm],
        out_specs=[hbm, hbm],
        out_shape=[
            jax.ShapeDtypeStruct(x.shape, x.dtype),
            jax.ShapeDtypeStruct(edge_attr.shape, edge_attr.dtype),
        ],
        scratch_shapes=[
            pltpu.MemorySpace.VMEM(x.shape, x.dtype),
            pltpu.MemorySpace.VMEM((2, n_edges // _NCHUNK, d_edge), edge_attr.dtype),
            pltpu.SemaphoreType.DMA,
            pltpu.SemaphoreType.DMA((2,)),
            pltpu.SemaphoreType.DMA((2,)),
        ],
    )(x, edge_attr)
    return (out[0], out[1])
